# baseline (speedup 1.0000x reference)
import functools

import jax
import jax.numpy as jnp
from jax import lax
from jax.experimental import pallas as pl
from jax.experimental.pallas import tpu as pltpu
from jax.experimental.pallas import tpu_sc as plsc

N = 10000
E = 320000
D_EDGE = 16
H = 128
OUT = 3
L = 4

NC, NS = 2, 16
NW = NC * NS
EW = E // NW
CCH = 80
NCHUNK = EW // CCH
NP_PAD = 10240
NPT = NP_PAD // NS

BE = 4000
BN = 2000

_EPS = 1e-5


def _sc_mesh():
    return plsc.VectorSubcoreMesh(
        core_axis_name="c", subcore_axis_name="s",
        num_cores=NC, num_subcores=NS)


GB = 5
NG = NCHUNK // GB


def _gather_body(table_hbm, idx_hbm, out_hbm, idx_v, *bufs):
    rows = bufs[:GB]
    gs = bufs[GB:2 * GB]
    os = bufs[2 * GB:3 * GB]
    wid = lax.axis_index("s") * NC + lax.axis_index("c")
    base = wid * EW
    pltpu.sync_copy(idx_hbm.at[pl.ds(base, EW)], idx_v)

    def group(j, carry):
        k0 = j * GB
        gcps = [
            pltpu.async_copy(
                table_hbm.at[idx_v.at[pl.ds((k0 + b) * CCH, CCH)]],
                rows[b], gs[b])
            for b in range(GB)
        ]
        ocps = []
        for b in range(GB):
            gcps[b].wait()
            ocps.append(pltpu.async_copy(
                rows[b], out_hbm.at[pl.ds(base + (k0 + b) * CCH, CCH)],
                os[b]))
        for cp in ocps:
            cp.wait()
        return carry

    lax.fori_loop(0, NG, group, 0)


@functools.lru_cache(maxsize=None)
def _make_gather(width):
    return functools.partial(
        pl.kernel,
        out_type=jax.ShapeDtypeStruct((E, width), jnp.float32),
        mesh=_sc_mesh(),
        scratch_types=[
            pltpu.VMEM((EW,), jnp.int32),
            *[pltpu.VMEM((CCH, width), jnp.float32) for _ in range(GB)],
            *[pltpu.SemaphoreType.DMA for _ in range(2 * GB)],
        ],
    )(_gather_body)


SGB = 1


def _scatter_body(m_hbm, idx_hbm, zeros_hbm, out_hbm, acc, *bufs):
    idxs = bufs[:SGB]
    rows = bufs[SGB:2 * SGB]
    isem = bufs[2 * SGB:3 * SGB]
    msem = bufs[3 * SGB:4 * SGB]
    c = lax.axis_index("c")
    s = lax.axis_index("s")
    pltpu.sync_copy(zeros_hbm.at[pl.ds(s * NPT, NPT)],
                    acc.at[pl.ds(s * NPT, NPT)])
    plsc.subcore_barrier()
    base = (c * NS + s) * EW

    def group(j, carry):
        k0 = j * SGB
        icps = [pltpu.async_copy(
            idx_hbm.at[pl.ds(base + (k0 + b) * CCH, CCH)], idxs[b], isem[b])
            for b in range(SGB)]
        mcps = [pltpu.async_copy(
            m_hbm.at[pl.ds(base + (k0 + b) * CCH, CCH)], rows[b], msem[b])
            for b in range(SGB)]
        for b in range(SGB):
            icps[b].wait()
            mcps[b].wait()
            pltpu.sync_copy(rows[b], acc.at[idxs[b]], add=True)
        return carry

    lax.fori_loop(0, NCHUNK // SGB, group, 0)
    plsc.subcore_barrier()
    pltpu.sync_copy(acc.at[pl.ds(s * NPT, NPT)],
                    out_hbm.at[c].at[pl.ds(s * NPT, NPT)])


@functools.lru_cache(maxsize=None)
def _make_scatter():
    return functools.partial(
        pl.kernel,
        out_type=jax.ShapeDtypeStruct((NC, NP_PAD, H), jnp.float32),
        mesh=_sc_mesh(),
        scratch_types=[
            pltpu.VMEM_SHARED((NP_PAD, H), jnp.float32),
            *[pltpu.VMEM((CCH,), jnp.int32) for _ in range(SGB)],
            *[pltpu.VMEM((CCH, H), jnp.float32) for _ in range(SGB)],
            *[pltpu.SemaphoreType.DMA for _ in range(2 * SGB)],
        ],
    )(_scatter_body)


def _deg_body(idx_hbm, zeros_hbm, ones_hbm, out_hbm, idx_v, rows_v, acc):
    c = lax.axis_index("c")
    s = lax.axis_index("s")
    pltpu.sync_copy(zeros_hbm.at[pl.ds(s * NPT, NPT)],
                    acc.at[pl.ds(s * NPT, NPT)])
    plsc.subcore_barrier()
    pltpu.sync_copy(ones_hbm, rows_v)
    base = (c * NS + s) * EW

    def body(k, carry):
        pltpu.sync_copy(idx_hbm.at[pl.ds(base + k * CCH, CCH)], idx_v)
        pltpu.sync_copy(rows_v, acc.at[idx_v], add=True)
        return carry

    lax.fori_loop(0, NCHUNK, body, 0)
    plsc.subcore_barrier()
    pltpu.sync_copy(acc.at[pl.ds(s * NPT, NPT)],
                    out_hbm.at[c].at[pl.ds(s * NPT, NPT)])


@functools.lru_cache(maxsize=None)
def _make_deg():
    return functools.partial(
        pl.kernel,
        out_type=jax.ShapeDtypeStruct((NC, NP_PAD, H), jnp.float32),
        mesh=_sc_mesh(),
        scratch_types=[
            pltpu.VMEM((CCH,), jnp.int32),
            pltpu.VMEM((CCH, H), jnp.float32),
            pltpu.VMEM_SHARED((NP_PAD, H), jnp.float32),
        ],
    )(_deg_body)


def _ln_rows(t, g, b):
    mu = jnp.mean(t, axis=-1, keepdims=True)
    var = jnp.mean((t - mu) ** 2, axis=-1, keepdims=True)
    return (t - mu) * lax.rsqrt(var + _EPS) * g + b


def _edge0_body(gd, gs, ea, eW1, eb1, eW2in, eb2in, We, eW2, eb2,
                nWe, nW2, nb2, eg, eb, e_new, m_out):
    ga = gd[:, :H].astype(jnp.float32)
    gc = gd[:, H:].astype(jnp.float32)
    gsv = gs[...]
    ev = jnp.maximum(jnp.dot(ea[...], eW1[...]) + eb1[...], 0.0)
    ev = jnp.dot(ev, eW2in[...]) + eb2in[...]
    z = jnp.maximum(ga + gsv + jnp.dot(ev, We[...]), 0.0)
    eupd = ev + jnp.dot(z, eW2[...]) + eb2[...]
    u = jnp.maximum(gc + jnp.dot(eupd, nWe[...]), 0.0)
    m_out[...] = jnp.dot(u, nW2[...]) + nb2[...]
    e_new[...] = _ln_rows(ev + eupd, eg[...], eb[...]).astype(jnp.bfloat16)


def _edge_body(gd, gs, e, We, eW2, eb2, nWe, nW2, nb2, eg, eb, e_new, m_out):
    ev = e[...].astype(jnp.float32)
    ga = gd[:, :H].astype(jnp.float32)
    gc = gd[:, H:].astype(jnp.float32)
    gsv = gs[...]
    z = jnp.maximum(ga + gsv + jnp.dot(ev, We[...]), 0.0)
    eupd = ev + jnp.dot(z, eW2[...]) + eb2[...]
    u = jnp.maximum(gc + jnp.dot(eupd, nWe[...]), 0.0)
    m_out[...] = jnp.dot(u, nW2[...]) + nb2[...]
    e_new[...] = _ln_rows(ev + eupd, eg[...], eb[...]).astype(jnp.bfloat16)


def _node_first_body(x, W1, b1, W2, b2, Wc, bc, h_out, td, ts):
    hv = jnp.maximum(jnp.dot(x[...], W1[...]) + b1[...], 0.0)
    hv = jnp.dot(hv, W2[...]) + b2[...]
    h_out[...] = hv
    t = jnp.dot(hv, Wc[...]) + bc[...]
    td[...] = t[:, :2 * H].astype(jnp.bfloat16)
    ts[...] = t[:, 2 * H:]


def _node_mid_body(h, sa, sb, dg, xg, xb, Wc, bc, h_out, td, ts):
    hv = h[...]
    hn = _ln_rows(hv + dg[...] * hv + sa[...] + sb[...], xg[...], xb[...])
    h_out[...] = hn
    t = jnp.dot(hn, Wc[...]) + bc[...]
    td[...] = t[:, :2 * H].astype(jnp.bfloat16)
    ts[...] = t[:, 2 * H:]


def _node_final_body(h, sa, sb, dg, xg, xb, W1, b1, W2p, b2p, out):
    hv = h[...]
    hn = _ln_rows(hv + dg[...] * hv + sa[...] + sb[...], xg[...], xb[...])
    u = jnp.maximum(jnp.dot(hn, W1[...]) + b1[...], 0.0)
    out[...] = jnp.dot(u, W2p[...]) + b2p[...]


def _bspec(bshape, const=False):
    if const:
        return pl.BlockSpec(bshape, lambda i: tuple(0 for _ in bshape))
    return pl.BlockSpec(bshape, lambda i: (i,) + tuple(0 for _ in bshape[1:]))


def _edge0_call(gd, gs, ea, eW1, eb1, eW2in, eb2in, We, eW2, eb2,
                nWe, nW2, nb2, eg, eb):
    return pl.pallas_call(
        _edge0_body,
        grid=(E // BE,),
        in_specs=[
            _bspec((BE, 2 * H)), _bspec((BE, H)), _bspec((BE, D_EDGE)),
            _bspec((D_EDGE, H), True), _bspec((1, H), True),
            _bspec((H, H), True), _bspec((1, H), True),
            _bspec((H, H), True),
            _bspec((H, H), True), _bspec((1, H), True),
            _bspec((H, H), True),
            _bspec((H, H), True), _bspec((1, H), True),
            _bspec((1, H), True), _bspec((1, H), True),
        ],
        out_specs=[_bspec((BE, H)), _bspec((BE, H))],
        out_shape=[
            jax.ShapeDtypeStruct((E, H), jnp.bfloat16),
            jax.ShapeDtypeStruct((E, H), jnp.float32),
        ],
    )(gd, gs, ea, eW1, eb1, eW2in, eb2in, We, eW2, eb2, nWe, nW2, nb2, eg, eb)


def _edge_call(gd, gs, e, We, eW2, eb2, nWe, nW2, nb2, eg, eb):
    return pl.pallas_call(
        _edge_body,
        grid=(E // BE,),
        in_specs=[
            _bspec((BE, 2 * H)), _bspec((BE, H)), _bspec((BE, H)),
            _bspec((H, H), True),
            _bspec((H, H), True), _bspec((1, H), True),
            _bspec((H, H), True),
            _bspec((H, H), True), _bspec((1, H), True),
            _bspec((1, H), True), _bspec((1, H), True),
        ],
        out_specs=[_bspec((BE, H)), _bspec((BE, H))],
        out_shape=[
            jax.ShapeDtypeStruct((E, H), jnp.bfloat16),
            jax.ShapeDtypeStruct((E, H), jnp.float32),
        ],
    )(gd, gs, e, We, eW2, eb2, nWe, nW2, nb2, eg, eb)


def _node_first_call(x, W1, b1, W2, b2, Wc, bc):
    return pl.pallas_call(
        _node_first_body,
        grid=(N // BN,),
        in_specs=[
            _bspec((BN, H)),
            _bspec((H, H), True), _bspec((1, H), True),
            _bspec((H, H), True), _bspec((1, H), True),
            _bspec((H, 3 * H), True), _bspec((1, 3 * H), True),
        ],
        out_specs=[_bspec((BN, H)), _bspec((BN, 2 * H)), _bspec((BN, H))],
        out_shape=[
            jax.ShapeDtypeStruct((N, H), jnp.float32),
            jax.ShapeDtypeStruct((N, 2 * H), jnp.bfloat16),
            jax.ShapeDtypeStruct((N, H), jnp.float32),
        ],
    )(x, W1, b1, W2, b2, Wc, bc)


def _node_mid_call(h, sa, sb, dg, xg, xb, Wc, bc):
    return pl.pallas_call(
        _node_mid_body,
        grid=(N // BN,),
        in_specs=[
            _bspec((BN, H)), _bspec((BN, H)), _bspec((BN, H)),
            _bspec((BN, 1)),
            _bspec((1, H), True), _bspec((1, H), True),
            _bspec((H, 3 * H), True), _bspec((1, 3 * H), True),
        ],
        out_specs=[_bspec((BN, H)), _bspec((BN, 2 * H)), _bspec((BN, H))],
        out_shape=[
            jax.ShapeDtypeStruct((N, H), jnp.float32),
            jax.ShapeDtypeStruct((N, 2 * H), jnp.bfloat16),
            jax.ShapeDtypeStruct((N, H), jnp.float32),
        ],
    )(h, sa, sb, dg, xg, xb, Wc, bc)


def _node_final_call(h, sa, sb, dg, xg, xb, W1, b1, W2p, b2p):
    return pl.pallas_call(
        _node_final_body,
        grid=(N // BN,),
        in_specs=[
            _bspec((BN, H)), _bspec((BN, H)), _bspec((BN, H)),
            _bspec((BN, 1)),
            _bspec((1, H), True), _bspec((1, H), True),
            _bspec((H, H), True), _bspec((1, H), True),
            _bspec((H, H), True), _bspec((1, H), True),
        ],
        out_specs=_bspec((BN, H)),
        out_shape=jax.ShapeDtypeStruct((N, H), jnp.float32),
    )(h, sa, sb, dg, xg, xb, W1, b1, W2p, b2p)


def kernel(x, edge_attr, edge_index, params):
    p = params
    src = edge_index[0].astype(jnp.int32)
    dst = edge_index[1].astype(jnp.int32)
    zeros_n = jnp.zeros((NP_PAD, H), jnp.float32)
    ones_c = jnp.ones((CCH, H), jnp.float32)

    def row(v):
        return v.reshape(1, -1)

    pd = _make_deg()(dst, zeros_n, ones_c)
    dg = (pd[0, :N, :1] + pd[1, :N, :1])

    Wc, bc = [], []
    for l in range(L):
        Wc.append(jnp.concatenate(
            [p['em_W1'][l][:H], p['nm_W1'][l][:H], p['em_W1'][l][H:2 * H]],
            axis=1))
        bc.append(jnp.concatenate(
            [p['em_b1'][l], p['nm_b1'][l], jnp.zeros((H,), jnp.float32)]
        ).reshape(1, 3 * H))

    de_W2p = jnp.zeros((H, H), jnp.float32).at[:, :OUT].set(p['de_W2'])
    de_b2p = jnp.zeros((1, H), jnp.float32).at[0, :OUT].set(p['de_b2'])

    h, td, ts = _node_first_call(
        x, p['ne_W1'], row(p['ne_b1']), p['ne_W2'], row(p['ne_b2']),
        Wc[0], bc[0])

    e = None
    for l in range(L):
        tdp = lax.bitcast_convert_type(
            td.reshape(N, H, 2), jnp.float32)
        gdp = _make_gather(H)(tdp, dst)
        gs = _make_gather(H)(ts, src)
        gd = lax.bitcast_convert_type(gdp, jnp.bfloat16).reshape(E, 2 * H)
        We = p['em_W1'][l][2 * H:]
        nWe = p['nm_W1'][l][H:]
        if l == 0:
            e, m = _edge0_call(
                gd, gs, edge_attr,
                p['ee_W1'], row(p['ee_b1']), p['ee_W2'], row(p['ee_b2']),
                We, p['em_W2'][l], row(p['em_b2'][l]),
                nWe, p['nm_W2'][l], row(p['nm_b2'][l]),
                row(p['eg'][l]), row(p['eb'][l]))
        else:
            e, m = _edge_call(
                gd, gs, e,
                We, p['em_W2'][l], row(p['em_b2'][l]),
                nWe, p['nm_W2'][l], row(p['nm_b2'][l]),
                row(p['eg'][l]), row(p['eb'][l]))
        sp = _make_scatter()(m, dst, zeros_n)
        sa, sb = sp[0, :N], sp[1, :N]
        if l < L - 1:
            h, td, ts = _node_mid_call(
                h, sa, sb, dg,
                row(p['xg'][l]), row(p['xb'][l]), Wc[l + 1], bc[l + 1])
        else:
            out = _node_final_call(
                h, sa, sb, dg,
                row(p['xg'][l]), row(p['xb'][l]),
                p['de_W1'], row(p['de_b1']), de_W2p, de_b2p)
    return out[:, :OUT]


# speedup vs baseline: 2.8625x; 2.8625x over previous
import functools

import jax
import jax.numpy as jnp
from jax import lax
from jax.experimental import pallas as pl
from jax.experimental.pallas import tpu as pltpu
from jax.experimental.pallas import tpu_sc as plsc

N = 10000
E = 320000
D_EDGE = 16
H = 128
OUT = 3
L = 4

NC, NS = 2, 16
NW = NC * NS
EW = E // NW
CCH = 80
NCHUNK = EW // CCH
NP_PAD = 10240
NPT = NP_PAD // NS

BE = 4000
BN = 2000

_EPS = 1e-5


def _sc_mesh():
    return plsc.VectorSubcoreMesh(
        core_axis_name="c", subcore_axis_name="s",
        num_cores=NC, num_subcores=NS)


GB = 5
NG = NCHUNK // GB


def _gather_body(table_hbm, idx_hbm, out_hbm, idx_v, *bufs):
    rows = bufs[:GB]
    gs = bufs[GB:2 * GB]
    os = bufs[2 * GB:3 * GB]
    wid = lax.axis_index("s") * NC + lax.axis_index("c")
    base = wid * EW
    pltpu.sync_copy(idx_hbm.at[pl.ds(base, EW)], idx_v)

    def group(j, carry):
        k0 = j * GB
        gcps = [
            pltpu.async_copy(
                table_hbm.at[idx_v.at[pl.ds((k0 + b) * CCH, CCH)]],
                rows[b], gs[b])
            for b in range(GB)
        ]
        ocps = []
        for b in range(GB):
            gcps[b].wait()
            ocps.append(pltpu.async_copy(
                rows[b], out_hbm.at[pl.ds(base + (k0 + b) * CCH, CCH)],
                os[b]))
        for cp in ocps:
            cp.wait()
        return carry

    lax.fori_loop(0, NG, group, 0)


@functools.lru_cache(maxsize=None)
def _make_gather(width):
    return functools.partial(
        pl.kernel,
        out_type=jax.ShapeDtypeStruct((E, width), jnp.float32),
        mesh=_sc_mesh(),
        scratch_types=[
            pltpu.VMEM((EW,), jnp.int32),
            *[pltpu.VMEM((CCH, width), jnp.float32) for _ in range(GB)],
            *[pltpu.SemaphoreType.DMA for _ in range(2 * GB)],
        ],
    )(_gather_body)


SGB = 1


def _scatter_body(m_hbm, idx_hbm, zeros_hbm, out_hbm, acc, *bufs):
    idxs = bufs[:SGB]
    rows = bufs[SGB:2 * SGB]
    isem = bufs[2 * SGB:3 * SGB]
    msem = bufs[3 * SGB:4 * SGB]
    c = lax.axis_index("c")
    s = lax.axis_index("s")
    pltpu.sync_copy(zeros_hbm.at[pl.ds(s * NPT, NPT)],
                    acc.at[pl.ds(s * NPT, NPT)])
    plsc.subcore_barrier()
    base = (c * NS + s) * EW

    def group(j, carry):
        k0 = j * SGB
        icps = [pltpu.async_copy(
            idx_hbm.at[pl.ds(base + (k0 + b) * CCH, CCH)], idxs[b], isem[b])
            for b in range(SGB)]
        mcps = [pltpu.async_copy(
            m_hbm.at[pl.ds(base + (k0 + b) * CCH, CCH)], rows[b], msem[b])
            for b in range(SGB)]
        for b in range(SGB):
            icps[b].wait()
            mcps[b].wait()
            pltpu.sync_copy(rows[b], acc.at[idxs[b]], add=True)
        return carry

    lax.fori_loop(0, NCHUNK // SGB, group, 0)
    plsc.subcore_barrier()
    pltpu.sync_copy(acc.at[pl.ds(s * NPT, NPT)],
                    out_hbm.at[c].at[pl.ds(s * NPT, NPT)])


@functools.lru_cache(maxsize=None)
def _make_scatter():
    return functools.partial(
        pl.kernel,
        out_type=jax.ShapeDtypeStruct((NC, NP_PAD, H), jnp.float32),
        mesh=_sc_mesh(),
        scratch_types=[
            pltpu.VMEM_SHARED((NP_PAD, H), jnp.float32),
            *[pltpu.VMEM((CCH,), jnp.int32) for _ in range(SGB)],
            *[pltpu.VMEM((CCH, H), jnp.float32) for _ in range(SGB)],
            *[pltpu.SemaphoreType.DMA for _ in range(2 * SGB)],
        ],
    )(_scatter_body)


def _deg_body(idx_hbm, zeros_hbm, ones_hbm, out_hbm, idx_v, rows_v, acc):
    c = lax.axis_index("c")
    s = lax.axis_index("s")
    pltpu.sync_copy(zeros_hbm.at[pl.ds(s * NPT, NPT)],
                    acc.at[pl.ds(s * NPT, NPT)])
    plsc.subcore_barrier()
    pltpu.sync_copy(ones_hbm, rows_v)
    base = (c * NS + s) * EW

    def body(k, carry):
        pltpu.sync_copy(idx_hbm.at[pl.ds(base + k * CCH, CCH)], idx_v)
        pltpu.sync_copy(rows_v, acc.at[idx_v], add=True)
        return carry

    lax.fori_loop(0, NCHUNK, body, 0)
    plsc.subcore_barrier()
    pltpu.sync_copy(acc.at[pl.ds(s * NPT, NPT)],
                    out_hbm.at[c].at[pl.ds(s * NPT, NPT)])


@functools.lru_cache(maxsize=None)
def _make_deg():
    return functools.partial(
        pl.kernel,
        out_type=jax.ShapeDtypeStruct((NC, NP_PAD, H), jnp.float32),
        mesh=_sc_mesh(),
        scratch_types=[
            pltpu.VMEM((CCH,), jnp.int32),
            pltpu.VMEM((CCH, H), jnp.float32),
            pltpu.VMEM_SHARED((NP_PAD, H), jnp.float32),
        ],
    )(_deg_body)


def _pack2(a, c):
    au = lax.bitcast_convert_type(a, jnp.uint32)
    cu = lax.bitcast_convert_type(c, jnp.uint32)
    w = ((au + 0x8000) & jnp.uint32(0xFFFF0000)) | ((cu + 0x8000) >> 16)
    return lax.bitcast_convert_type(w, jnp.float32)


def _unpack2(p):
    w = lax.bitcast_convert_type(p, jnp.uint32)
    a = lax.bitcast_convert_type(w & jnp.uint32(0xFFFF0000), jnp.float32)
    c = lax.bitcast_convert_type(w << 16, jnp.float32)
    return a, c


def _ln_rows(t, g, b):
    mu = jnp.mean(t, axis=-1, keepdims=True)
    var = jnp.mean((t - mu) ** 2, axis=-1, keepdims=True)
    return (t - mu) * lax.rsqrt(var + _EPS) * g + b


def _edge0_body(gd, gs, ea, eW1, eb1, eW2in, eb2in, We, eW2, eb2,
                nWe, nW2, nb2, eg, eb, e_new, m_out):
    ga, gc = _unpack2(gd[...])
    gsv = gs[...]
    ev = jnp.maximum(jnp.dot(ea[...], eW1[...]) + eb1[...], 0.0)
    ev = jnp.dot(ev, eW2in[...]) + eb2in[...]
    z = jnp.maximum(ga + gsv + jnp.dot(ev, We[...]), 0.0)
    eupd = ev + jnp.dot(z, eW2[...]) + eb2[...]
    u = jnp.maximum(gc + jnp.dot(eupd, nWe[...]), 0.0)
    m_out[...] = jnp.dot(u, nW2[...]) + nb2[...]
    e_new[...] = _ln_rows(ev + eupd, eg[...], eb[...]).astype(jnp.bfloat16)


def _edge_body(gd, gs, e, We, eW2, eb2, nWe, nW2, nb2, eg, eb, e_new, m_out):
    ev = e[...].astype(jnp.float32)
    ga, gc = _unpack2(gd[...])
    gsv = gs[...]
    z = jnp.maximum(ga + gsv + jnp.dot(ev, We[...]), 0.0)
    eupd = ev + jnp.dot(z, eW2[...]) + eb2[...]
    u = jnp.maximum(gc + jnp.dot(eupd, nWe[...]), 0.0)
    m_out[...] = jnp.dot(u, nW2[...]) + nb2[...]
    e_new[...] = _ln_rows(ev + eupd, eg[...], eb[...]).astype(jnp.bfloat16)


def _node_first_body(x, W1, b1, W2, b2, Wc, bc, h_out, td, ts):
    hv = jnp.maximum(jnp.dot(x[...], W1[...]) + b1[...], 0.0)
    hv = jnp.dot(hv, W2[...]) + b2[...]
    h_out[...] = hv
    t = jnp.dot(hv, Wc[...]) + bc[...]
    td[...] = _pack2(t[:, :H], t[:, H:2 * H])
    ts[...] = t[:, 2 * H:]


def _node_mid_body(h, sa, sb, dg, xg, xb, Wc, bc, h_out, td, ts):
    hv = h[...]
    hn = _ln_rows(hv + dg[...] * hv + sa[...] + sb[...], xg[...], xb[...])
    h_out[...] = hn
    t = jnp.dot(hn, Wc[...]) + bc[...]
    td[...] = _pack2(t[:, :H], t[:, H:2 * H])
    ts[...] = t[:, 2 * H:]


def _node_final_body(h, sa, sb, dg, xg, xb, W1, b1, W2p, b2p, out):
    hv = h[...]
    hn = _ln_rows(hv + dg[...] * hv + sa[...] + sb[...], xg[...], xb[...])
    u = jnp.maximum(jnp.dot(hn, W1[...]) + b1[...], 0.0)
    out[...] = jnp.dot(u, W2p[...]) + b2p[...]


def _bspec(bshape, const=False):
    if const:
        return pl.BlockSpec(bshape, lambda i: tuple(0 for _ in bshape))
    return pl.BlockSpec(bshape, lambda i: (i,) + tuple(0 for _ in bshape[1:]))


def _edge0_call(gd, gs, ea, eW1, eb1, eW2in, eb2in, We, eW2, eb2,
                nWe, nW2, nb2, eg, eb):
    return pl.pallas_call(
        _edge0_body,
        grid=(E // BE,),
        in_specs=[
            _bspec((BE, H)), _bspec((BE, H)), _bspec((BE, D_EDGE)),
            _bspec((D_EDGE, H), True), _bspec((1, H), True),
            _bspec((H, H), True), _bspec((1, H), True),
            _bspec((H, H), True),
            _bspec((H, H), True), _bspec((1, H), True),
            _bspec((H, H), True),
            _bspec((H, H), True), _bspec((1, H), True),
            _bspec((1, H), True), _bspec((1, H), True),
        ],
        out_specs=[_bspec((BE, H)), _bspec((BE, H))],
        out_shape=[
            jax.ShapeDtypeStruct((E, H), jnp.bfloat16),
            jax.ShapeDtypeStruct((E, H), jnp.float32),
        ],
    )(gd, gs, ea, eW1, eb1, eW2in, eb2in, We, eW2, eb2, nWe, nW2, nb2, eg, eb)


def _edge_call(gd, gs, e, We, eW2, eb2, nWe, nW2, nb2, eg, eb):
    return pl.pallas_call(
        _edge_body,
        grid=(E // BE,),
        in_specs=[
            _bspec((BE, H)), _bspec((BE, H)), _bspec((BE, H)),
            _bspec((H, H), True),
            _bspec((H, H), True), _bspec((1, H), True),
            _bspec((H, H), True),
            _bspec((H, H), True), _bspec((1, H), True),
            _bspec((1, H), True), _bspec((1, H), True),
        ],
        out_specs=[_bspec((BE, H)), _bspec((BE, H))],
        out_shape=[
            jax.ShapeDtypeStruct((E, H), jnp.bfloat16),
            jax.ShapeDtypeStruct((E, H), jnp.float32),
        ],
    )(gd, gs, e, We, eW2, eb2, nWe, nW2, nb2, eg, eb)


def _node_first_call(x, W1, b1, W2, b2, Wc, bc):
    return pl.pallas_call(
        _node_first_body,
        grid=(N // BN,),
        in_specs=[
            _bspec((BN, H)),
            _bspec((H, H), True), _bspec((1, H), True),
            _bspec((H, H), True), _bspec((1, H), True),
            _bspec((H, 3 * H), True), _bspec((1, 3 * H), True),
        ],
        out_specs=[_bspec((BN, H)), _bspec((BN, H)), _bspec((BN, H))],
        out_shape=[
            jax.ShapeDtypeStruct((N, H), jnp.float32),
            jax.ShapeDtypeStruct((N, H), jnp.float32),
            jax.ShapeDtypeStruct((N, H), jnp.float32),
        ],
    )(x, W1, b1, W2, b2, Wc, bc)


def _node_mid_call(h, sa, sb, dg, xg, xb, Wc, bc):
    return pl.pallas_call(
        _node_mid_body,
        grid=(N // BN,),
        in_specs=[
            _bspec((BN, H)), _bspec((BN, H)), _bspec((BN, H)),
            _bspec((BN, 1)),
            _bspec((1, H), True), _bspec((1, H), True),
            _bspec((H, 3 * H), True), _bspec((1, 3 * H), True),
        ],
        out_specs=[_bspec((BN, H)), _bspec((BN, H)), _bspec((BN, H))],
        out_shape=[
            jax.ShapeDtypeStruct((N, H), jnp.float32),
            jax.ShapeDtypeStruct((N, H), jnp.float32),
            jax.ShapeDtypeStruct((N, H), jnp.float32),
        ],
    )(h, sa, sb, dg, xg, xb, Wc, bc)


def _node_final_call(h, sa, sb, dg, xg, xb, W1, b1, W2p, b2p):
    return pl.pallas_call(
        _node_final_body,
        grid=(N // BN,),
        in_specs=[
            _bspec((BN, H)), _bspec((BN, H)), _bspec((BN, H)),
            _bspec((BN, 1)),
            _bspec((1, H), True), _bspec((1, H), True),
            _bspec((H, H), True), _bspec((1, H), True),
            _bspec((H, H), True), _bspec((1, H), True),
        ],
        out_specs=_bspec((BN, H)),
        out_shape=jax.ShapeDtypeStruct((N, H), jnp.float32),
    )(h, sa, sb, dg, xg, xb, W1, b1, W2p, b2p)


def kernel(x, edge_attr, edge_index, params):
    p = params
    src = edge_index[0].astype(jnp.int32)
    dst = edge_index[1].astype(jnp.int32)
    zeros_n = jnp.zeros((NP_PAD, H), jnp.float32)
    ones_c = jnp.ones((CCH, H), jnp.float32)

    def row(v):
        return v.reshape(1, -1)

    pd = _make_deg()(dst, zeros_n, ones_c)
    dg = (pd[0, :N, :1] + pd[1, :N, :1])

    Wc, bc = [], []
    for l in range(L):
        Wc.append(jnp.concatenate(
            [p['em_W1'][l][:H], p['nm_W1'][l][:H], p['em_W1'][l][H:2 * H]],
            axis=1))
        bc.append(jnp.concatenate(
            [p['em_b1'][l], p['nm_b1'][l], jnp.zeros((H,), jnp.float32)]
        ).reshape(1, 3 * H))

    de_W2p = jnp.zeros((H, H), jnp.float32).at[:, :OUT].set(p['de_W2'])
    de_b2p = jnp.zeros((1, H), jnp.float32).at[0, :OUT].set(p['de_b2'])

    h, td, ts = _node_first_call(
        x, p['ne_W1'], row(p['ne_b1']), p['ne_W2'], row(p['ne_b2']),
        Wc[0], bc[0])

    e = None
    for l in range(L):
        gd = _make_gather(H)(td, dst)
        gs = _make_gather(H)(ts, src)
        We = p['em_W1'][l][2 * H:]
        nWe = p['nm_W1'][l][H:]
        if l == 0:
            e, m = _edge0_call(
                gd, gs, edge_attr,
                p['ee_W1'], row(p['ee_b1']), p['ee_W2'], row(p['ee_b2']),
                We, p['em_W2'][l], row(p['em_b2'][l]),
                nWe, p['nm_W2'][l], row(p['nm_b2'][l]),
                row(p['eg'][l]), row(p['eb'][l]))
        else:
            e, m = _edge_call(
                gd, gs, e,
                We, p['em_W2'][l], row(p['em_b2'][l]),
                nWe, p['nm_W2'][l], row(p['nm_b2'][l]),
                row(p['eg'][l]), row(p['eb'][l]))
        sp = _make_scatter()(m, dst, zeros_n)
        sa, sb = sp[0, :N], sp[1, :N]
        if l < L - 1:
            h, td, ts = _node_mid_call(
                h, sa, sb, dg,
                row(p['xg'][l]), row(p['xb'][l]), Wc[l + 1], bc[l + 1])
        else:
            out = _node_final_call(
                h, sa, sb, dg,
                row(p['xg'][l]), row(p['xb'][l]),
                p['de_W1'], row(p['de_b1']), de_W2p, de_b2p)
    return out[:, :OUT]


# speedup vs baseline: 3.2729x; 1.1434x over previous
import functools

import jax
import jax.numpy as jnp
from jax import lax
from jax.experimental import pallas as pl
from jax.experimental.pallas import tpu as pltpu
from jax.experimental.pallas import tpu_sc as plsc

N = 10000
E = 320000
D_EDGE = 16
H = 128
OUT = 3
L = 4

NC, NS = 2, 16
NW = NC * NS
EW = E // NW
CCH = 80
NCHUNK = EW // CCH
NP_PAD = 10240
NPT = NP_PAD // NS

BE = 4000
BN = 2000

_EPS = 1e-5


def _sc_mesh():
    return plsc.VectorSubcoreMesh(
        core_axis_name="c", subcore_axis_name="s",
        num_cores=NC, num_subcores=NS)


GB = 5
NG = NCHUNK // GB


def _gather_body(table_hbm, idx_hbm, out_hbm, idx_v, rows_all, *sems):
    gsems = sems[:GB]
    osems = sems[GB:2 * GB]
    wid = lax.axis_index("s") * NC + lax.axis_index("c")
    base = wid * EW
    pltpu.sync_copy(idx_hbm.at[pl.ds(base, EW)], idx_v)

    def slot(p, b):
        return rows_all.at[pl.ds((p * GB + b) * CCH, CCH)]

    def gissue(k, p, b):
        return pltpu.async_copy(
            table_hbm.at[idx_v.at[pl.ds(k * CCH, CCH)]], slot(p, b),
            gsems[b])

    def gwait(p, b):
        pltpu.make_async_copy(
            table_hbm.at[idx_v.at[pl.ds(0, CCH)]], slot(p, b),
            gsems[b]).wait()

    def oissue(k, p, b):
        return pltpu.async_copy(
            slot(p, b), out_hbm.at[pl.ds(base + k * CCH, CCH)], osems[b])

    def owait(p, b):
        pltpu.make_async_copy(
            slot(p, b), out_hbm.at[pl.ds(base, CCH)], osems[b]).wait()

    for b in range(GB):
        gissue(b, 0, b)

    def body(j, carry):
        p = lax.rem(j, 2)
        q = 1 - p
        k0 = j * GB
        for b in range(GB):
            gwait(p, b)

            @pl.when(j > 0)
            def _():
                owait(q, b)

            gissue(k0 + GB + b, q, b)
            oissue(k0 + b, p, b)
        return carry

    lax.fori_loop(0, NG - 1, body, 0)

    pL = (NG - 1) % 2
    k0 = (NG - 1) * GB
    for b in range(GB):
        gwait(pL, b)
        owait(1 - pL, b)
        oissue(k0 + b, pL, b)
    for b in range(GB):
        owait(pL, b)


@functools.lru_cache(maxsize=None)
def _make_gather(width):
    return functools.partial(
        pl.kernel,
        out_type=jax.ShapeDtypeStruct((E, width), jnp.float32),
        mesh=_sc_mesh(),
        scratch_types=[
            pltpu.VMEM((EW,), jnp.int32),
            pltpu.VMEM((2 * GB * CCH, width), jnp.float32),
            *[pltpu.SemaphoreType.DMA for _ in range(2 * GB)],
        ],
    )(_gather_body)


def _scatter_body(m_hbm, idx3_hbm, zeros_hbm, out_hbm, acc, idx_all,
                  rows_all, msem):
    c = lax.axis_index("c")
    s = lax.axis_index("s")
    pltpu.sync_copy(zeros_hbm.at[pl.ds(s * NPT, NPT)],
                    acc.at[pl.ds(s * NPT, NPT)])
    plsc.subcore_barrier()
    wid = c * NS + s
    base = wid * EW
    pltpu.sync_copy(idx3_hbm.at[wid], idx_all)

    def slot(p):
        return rows_all.at[pl.ds(p * CCH, CCH)]

    def missue(k, p, sem):
        return pltpu.async_copy(
            m_hbm.at[pl.ds(base + k * CCH, CCH)], slot(p), sem)

    def mwait(p, sem):
        pltpu.make_async_copy(
            m_hbm.at[pl.ds(base, CCH)], slot(p), sem).wait()

    missue(0, 0, msem.at[0])
    missue(1, 1, msem.at[1])

    def body(k, carry):
        p = lax.rem(k, 2)
        mwait(p, msem.at[p])
        pltpu.sync_copy(slot(p), acc.at[idx_all.at[k]], add=True)

        @pl.when(k < NCHUNK - 2)
        def _():
            missue(k + 2, p, msem.at[p])

        return carry

    lax.fori_loop(0, NCHUNK, body, 0)
    plsc.subcore_barrier()
    pltpu.sync_copy(acc.at[pl.ds(s * NPT, NPT)],
                    out_hbm.at[c].at[pl.ds(s * NPT, NPT)])


@functools.lru_cache(maxsize=None)
def _make_scatter():
    return functools.partial(
        pl.kernel,
        out_type=jax.ShapeDtypeStruct((NC, NP_PAD, H), jnp.float32),
        mesh=_sc_mesh(),
        scratch_types=[
            pltpu.VMEM_SHARED((NP_PAD, H), jnp.float32),
            pltpu.VMEM((NCHUNK, CCH), jnp.int32),
            pltpu.VMEM((2 * CCH, H), jnp.float32),
            pltpu.SemaphoreType.DMA((2,)),
        ],
    )(_scatter_body)


def _deg_body(idx_hbm, zeros_hbm, ones_hbm, out_hbm, idx_v, rows_v, acc):
    c = lax.axis_index("c")
    s = lax.axis_index("s")
    pltpu.sync_copy(zeros_hbm.at[pl.ds(s * NPT, NPT)],
                    acc.at[pl.ds(s * NPT, NPT)])
    plsc.subcore_barrier()
    pltpu.sync_copy(ones_hbm, rows_v)
    base = (c * NS + s) * EW

    def body(k, carry):
        pltpu.sync_copy(idx_hbm.at[pl.ds(base + k * CCH, CCH)], idx_v)
        pltpu.sync_copy(rows_v, acc.at[idx_v], add=True)
        return carry

    lax.fori_loop(0, NCHUNK, body, 0)
    plsc.subcore_barrier()
    pltpu.sync_copy(acc.at[pl.ds(s * NPT, NPT)],
                    out_hbm.at[c].at[pl.ds(s * NPT, NPT)])


@functools.lru_cache(maxsize=None)
def _make_deg():
    return functools.partial(
        pl.kernel,
        out_type=jax.ShapeDtypeStruct((NC, NP_PAD, H), jnp.float32),
        mesh=_sc_mesh(),
        scratch_types=[
            pltpu.VMEM((CCH,), jnp.int32),
            pltpu.VMEM((CCH, H), jnp.float32),
            pltpu.VMEM_SHARED((NP_PAD, H), jnp.float32),
        ],
    )(_deg_body)


def _pack2(a, c):
    au = lax.bitcast_convert_type(a, jnp.uint32)
    cu = lax.bitcast_convert_type(c, jnp.uint32)
    w = ((au + 0x8000) & jnp.uint32(0xFFFF0000)) | ((cu + 0x8000) >> 16)
    return lax.bitcast_convert_type(w, jnp.float32)


def _unpack2(p):
    w = lax.bitcast_convert_type(p, jnp.uint32)
    a = lax.bitcast_convert_type(w & jnp.uint32(0xFFFF0000), jnp.float32)
    c = lax.bitcast_convert_type(w << 16, jnp.float32)
    return a, c


def _ln_rows(t, g, b):
    mu = jnp.mean(t, axis=-1, keepdims=True)
    var = jnp.mean((t - mu) ** 2, axis=-1, keepdims=True)
    return (t - mu) * lax.rsqrt(var + _EPS) * g + b


def _edge0_body(gd, gs, ea, eW1, eb1, eW2in, eb2in, We, eW2, eb2,
                nWe, nW2, nb2, eg, eb, e_new, m_out):
    ga, gc = _unpack2(gd[...])
    gsv = gs[...]
    ev = jnp.maximum(jnp.dot(ea[...], eW1[...]) + eb1[...], 0.0)
    ev = jnp.dot(ev, eW2in[...]) + eb2in[...]
    z = jnp.maximum(ga + gsv + jnp.dot(ev, We[...]), 0.0)
    eupd = ev + jnp.dot(z, eW2[...]) + eb2[...]
    u = jnp.maximum(gc + jnp.dot(eupd, nWe[...]), 0.0)
    m_out[...] = jnp.dot(u, nW2[...]) + nb2[...]
    e_new[...] = _ln_rows(ev + eupd, eg[...], eb[...]).astype(jnp.bfloat16)


def _edge_body(gd, gs, e, We, eW2, eb2, nWe, nW2, nb2, eg, eb, e_new, m_out):
    ev = e[...].astype(jnp.float32)
    ga, gc = _unpack2(gd[...])
    gsv = gs[...]
    z = jnp.maximum(ga + gsv + jnp.dot(ev, We[...]), 0.0)
    eupd = ev + jnp.dot(z, eW2[...]) + eb2[...]
    u = jnp.maximum(gc + jnp.dot(eupd, nWe[...]), 0.0)
    m_out[...] = jnp.dot(u, nW2[...]) + nb2[...]
    e_new[...] = _ln_rows(ev + eupd, eg[...], eb[...]).astype(jnp.bfloat16)


def _node_first_body(x, W1, b1, W2, b2, Wc, bc, h_out, td, ts):
    hv = jnp.maximum(jnp.dot(x[...], W1[...]) + b1[...], 0.0)
    hv = jnp.dot(hv, W2[...]) + b2[...]
    h_out[...] = hv
    t = jnp.dot(hv, Wc[...]) + bc[...]
    td[...] = _pack2(t[:, :H], t[:, H:2 * H])
    ts[...] = t[:, 2 * H:]


def _node_mid_body(h, sa, sb, dg, xg, xb, Wc, bc, h_out, td, ts):
    hv = h[...]
    hn = _ln_rows(hv + dg[...] * hv + sa[...] + sb[...], xg[...], xb[...])
    h_out[...] = hn
    t = jnp.dot(hn, Wc[...]) + bc[...]
    td[...] = _pack2(t[:, :H], t[:, H:2 * H])
    ts[...] = t[:, 2 * H:]


def _node_final_body(h, sa, sb, dg, xg, xb, W1, b1, W2p, b2p, out):
    hv = h[...]
    hn = _ln_rows(hv + dg[...] * hv + sa[...] + sb[...], xg[...], xb[...])
    u = jnp.maximum(jnp.dot(hn, W1[...]) + b1[...], 0.0)
    out[...] = jnp.dot(u, W2p[...]) + b2p[...]


def _bspec(bshape, const=False):
    if const:
        return pl.BlockSpec(bshape, lambda i: tuple(0 for _ in bshape))
    return pl.BlockSpec(bshape, lambda i: (i,) + tuple(0 for _ in bshape[1:]))


def _edge0_call(gd, gs, ea, eW1, eb1, eW2in, eb2in, We, eW2, eb2,
                nWe, nW2, nb2, eg, eb):
    return pl.pallas_call(
        _edge0_body,
        grid=(E // BE,),
        in_specs=[
            _bspec((BE, H)), _bspec((BE, H)), _bspec((BE, D_EDGE)),
            _bspec((D_EDGE, H), True), _bspec((1, H), True),
            _bspec((H, H), True), _bspec((1, H), True),
            _bspec((H, H), True),
            _bspec((H, H), True), _bspec((1, H), True),
            _bspec((H, H), True),
            _bspec((H, H), True), _bspec((1, H), True),
            _bspec((1, H), True), _bspec((1, H), True),
        ],
        out_specs=[_bspec((BE, H)), _bspec((BE, H))],
        out_shape=[
            jax.ShapeDtypeStruct((E, H), jnp.bfloat16),
            jax.ShapeDtypeStruct((E, H), jnp.float32),
        ],
    )(gd, gs, ea, eW1, eb1, eW2in, eb2in, We, eW2, eb2, nWe, nW2, nb2, eg, eb)


def _edge_call(gd, gs, e, We, eW2, eb2, nWe, nW2, nb2, eg, eb):
    return pl.pallas_call(
        _edge_body,
        grid=(E // BE,),
        in_specs=[
            _bspec((BE, H)), _bspec((BE, H)), _bspec((BE, H)),
            _bspec((H, H), True),
            _bspec((H, H), True), _bspec((1, H), True),
            _bspec((H, H), True),
            _bspec((H, H), True), _bspec((1, H), True),
            _bspec((1, H), True), _bspec((1, H), True),
        ],
        out_specs=[_bspec((BE, H)), _bspec((BE, H))],
        out_shape=[
            jax.ShapeDtypeStruct((E, H), jnp.bfloat16),
            jax.ShapeDtypeStruct((E, H), jnp.float32),
        ],
    )(gd, gs, e, We, eW2, eb2, nWe, nW2, nb2, eg, eb)


def _node_first_call(x, W1, b1, W2, b2, Wc, bc):
    return pl.pallas_call(
        _node_first_body,
        grid=(N // BN,),
        in_specs=[
            _bspec((BN, H)),
            _bspec((H, H), True), _bspec((1, H), True),
            _bspec((H, H), True), _bspec((1, H), True),
            _bspec((H, 3 * H), True), _bspec((1, 3 * H), True),
        ],
        out_specs=[_bspec((BN, H)), _bspec((BN, H)), _bspec((BN, H))],
        out_shape=[
            jax.ShapeDtypeStruct((N, H), jnp.float32),
            jax.ShapeDtypeStruct((N, H), jnp.float32),
            jax.ShapeDtypeStruct((N, H), jnp.float32),
        ],
    )(x, W1, b1, W2, b2, Wc, bc)


def _node_mid_call(h, sa, sb, dg, xg, xb, Wc, bc):
    return pl.pallas_call(
        _node_mid_body,
        grid=(N // BN,),
        in_specs=[
            _bspec((BN, H)), _bspec((BN, H)), _bspec((BN, H)),
            _bspec((BN, 1)),
            _bspec((1, H), True), _bspec((1, H), True),
            _bspec((H, 3 * H), True), _bspec((1, 3 * H), True),
        ],
        out_specs=[_bspec((BN, H)), _bspec((BN, H)), _bspec((BN, H))],
        out_shape=[
            jax.ShapeDtypeStruct((N, H), jnp.float32),
            jax.ShapeDtypeStruct((N, H), jnp.float32),
            jax.ShapeDtypeStruct((N, H), jnp.float32),
        ],
    )(h, sa, sb, dg, xg, xb, Wc, bc)


def _node_final_call(h, sa, sb, dg, xg, xb, W1, b1, W2p, b2p):
    return pl.pallas_call(
        _node_final_body,
        grid=(N // BN,),
        in_specs=[
            _bspec((BN, H)), _bspec((BN, H)), _bspec((BN, H)),
            _bspec((BN, 1)),
            _bspec((1, H), True), _bspec((1, H), True),
            _bspec((H, H), True), _bspec((1, H), True),
            _bspec((H, H), True), _bspec((1, H), True),
        ],
        out_specs=_bspec((BN, H)),
        out_shape=jax.ShapeDtypeStruct((N, H), jnp.float32),
    )(h, sa, sb, dg, xg, xb, W1, b1, W2p, b2p)


def kernel(x, edge_attr, edge_index, params):
    p = params
    src = edge_index[0].astype(jnp.int32)
    dst = edge_index[1].astype(jnp.int32)
    zeros_n = jnp.zeros((NP_PAD, H), jnp.float32)
    ones_c = jnp.ones((CCH, H), jnp.float32)
    dst3 = dst.reshape(NW, NCHUNK, CCH)

    def row(v):
        return v.reshape(1, -1)

    pd = _make_deg()(dst, zeros_n, ones_c)
    dg = (pd[0, :N, :1] + pd[1, :N, :1])

    Wc, bc = [], []
    for l in range(L):
        Wc.append(jnp.concatenate(
            [p['em_W1'][l][:H], p['nm_W1'][l][:H], p['em_W1'][l][H:2 * H]],
            axis=1))
        bc.append(jnp.concatenate(
            [p['em_b1'][l], p['nm_b1'][l], jnp.zeros((H,), jnp.float32)]
        ).reshape(1, 3 * H))

    de_W2p = jnp.zeros((H, H), jnp.float32).at[:, :OUT].set(p['de_W2'])
    de_b2p = jnp.zeros((1, H), jnp.float32).at[0, :OUT].set(p['de_b2'])

    h, td, ts = _node_first_call(
        x, p['ne_W1'], row(p['ne_b1']), p['ne_W2'], row(p['ne_b2']),
        Wc[0], bc[0])

    e = None
    for l in range(L):
        gd = _make_gather(H)(td, dst)
        gs = _make_gather(H)(ts, src)
        We = p['em_W1'][l][2 * H:]
        nWe = p['nm_W1'][l][H:]
        if l == 0:
            e, m = _edge0_call(
                gd, gs, edge_attr,
                p['ee_W1'], row(p['ee_b1']), p['ee_W2'], row(p['ee_b2']),
                We, p['em_W2'][l], row(p['em_b2'][l]),
                nWe, p['nm_W2'][l], row(p['nm_b2'][l]),
                row(p['eg'][l]), row(p['eb'][l]))
        else:
            e, m = _edge_call(
                gd, gs, e,
                We, p['em_W2'][l], row(p['em_b2'][l]),
                nWe, p['nm_W2'][l], row(p['nm_b2'][l]),
                row(p['eg'][l]), row(p['eb'][l]))
        sp = _make_scatter()(m, dst3, zeros_n)
        sa, sb = sp[0, :N], sp[1, :N]
        if l < L - 1:
            h, td, ts = _node_mid_call(
                h, sa, sb, dg,
                row(p['xg'][l]), row(p['xb'][l]), Wc[l + 1], bc[l + 1])
        else:
            out = _node_final_call(
                h, sa, sb, dg,
                row(p['xg'][l]), row(p['xb'][l]),
                p['de_W1'], row(p['de_b1']), de_W2p, de_b2p)
    return out[:, :OUT]
